# Initial kernel scaffold; baseline (speedup 1.0000x reference)
import jax
import jax.numpy as jnp
from jax.experimental import pallas as pl


def kernel(obs, w_gate, W1, b1, W2, b2):
    raise NotImplementedError("write your pallas kernel here")


# speedup vs baseline: 1.1196x; 1.1196x over previous
import jax
import jax.numpy as jnp
from jax.experimental import pallas as pl

_E = 8
_K = 4
_BM = 256


def _moe_dense_body(obs_ref, wg_ref, W1_ref, b1_ref, W2_ref, b2_ref, y_ref):
    x = obs_ref[...]
    logits = jnp.dot(x, wg_ref[...], preferred_element_type=jnp.float32)
    m = logits
    for _ in range(_K - 1):
        rmax = jnp.max(m, axis=1, keepdims=True)
        m = jnp.where(m == rmax, -jnp.inf, m)
    thresh = jnp.max(m, axis=1, keepdims=True)
    sel = logits >= thresh
    z = jnp.where(sel, logits, -jnp.inf)
    z = z - jnp.max(z, axis=1, keepdims=True)
    g = jnp.exp(z)
    g = g / jnp.sum(g, axis=1, keepdims=True)

    acc = jnp.zeros((x.shape[0], W2_ref.shape[2]), dtype=jnp.float32)
    for e in range(_E):
        h = jnp.dot(x, W1_ref[e], preferred_element_type=jnp.float32)
        h = jnp.maximum(h + b1_ref[e][None, :], 0.0)
        o = jnp.dot(h.astype(jnp.bfloat16), W2_ref[e],
                    preferred_element_type=jnp.float32)
        o = o + b2_ref[e][None, :]
        o = o - jnp.max(o, axis=1, keepdims=True)
        p = jnp.exp(o)
        p = p / jnp.sum(p, axis=1, keepdims=True)
        acc = acc + g[:, e:e + 1] * p
    y_ref[...] = acc


def kernel(obs, w_gate, W1, b1, W2, b2):
    B, D = obs.shape
    H = W1.shape[2]
    A = W2.shape[2]
    obs_bf = obs.astype(jnp.bfloat16)
    wg_bf = w_gate.astype(jnp.bfloat16)
    W1_bf = W1.astype(jnp.bfloat16)
    W2_bf = W2.astype(jnp.bfloat16)
    return pl.pallas_call(
        _moe_dense_body,
        grid=(B // _BM,),
        in_specs=[
            pl.BlockSpec((_BM, D), lambda i: (i, 0)),
            pl.BlockSpec((D, _E), lambda i: (0, 0)),
            pl.BlockSpec((_E, D, H), lambda i: (0, 0, 0)),
            pl.BlockSpec((_E, H), lambda i: (0, 0)),
            pl.BlockSpec((_E, H, A), lambda i: (0, 0, 0)),
            pl.BlockSpec((_E, A), lambda i: (0, 0)),
        ],
        out_specs=pl.BlockSpec((_BM, A), lambda i: (i, 0)),
        out_shape=jax.ShapeDtypeStruct((B, A), jnp.float32),
    )(obs_bf, wg_bf, W1_bf, b1, W2_bf, b2)
